# Initial kernel scaffold; baseline (speedup 1.0000x reference)
import jax
import jax.numpy as jnp
from jax.experimental import pallas as pl


def kernel(n, rel_bias_table):
    raise NotImplementedError("write your pallas kernel here")


# speedup vs baseline: 34.1882x; 34.1882x over previous
import math

import jax
import jax.numpy as jnp
from jax.experimental import pallas as pl
from jax.experimental.pallas import tpu as pltpu

N = 2048
H = 16
BI = 128
NJ = N // 128
NBLK = 2 * (NJ - 1) + 3


def _build_body(tbl_ref, strip_ref):
    h = pl.program_id(0)
    c_lo = tbl_ref[15, h]
    c_hi = tbl_ref[31, h]
    strip_ref[0, :, 0:NJ - 1, :] = jnp.full((BI, NJ - 1, 128), c_lo, jnp.float32)
    strip_ref[0, :, NJ + 2:NBLK, :] = jnp.full((BI, NJ - 1, 128), c_hi, jnp.float32)
    r = jax.lax.broadcasted_iota(jnp.int32, (BI, 3, 128), 0)
    t = jax.lax.broadcasted_iota(jnp.int32, (BI, 3, 128), 1)
    c = jax.lax.broadcasted_iota(jnp.int32, (BI, 3, 128), 2)
    rel = (c + (t - 1) * 128) - r
    nn = -rel
    ret = (nn < 0).astype(jnp.int32) * 16
    na = jnp.abs(nn)
    is_small = na < 8
    vl = 8 + (
        jnp.log(na.astype(jnp.float32) / 8.0) / math.log(16.0) * 8.0
    ).astype(jnp.int32)
    vl = jnp.minimum(vl, jnp.full_like(vl, 15))
    bucket = ret + jnp.where(is_small, na, vl)
    acc = jnp.zeros((BI, 3, 128), jnp.float32)
    for k in range(32):
        acc = acc + jnp.where(bucket == k, tbl_ref[k, h], 0.0)
    strip_ref[0, :, NJ - 1:NJ + 2, :] = acc


def _expand_body(strip_ref, out_ref):
    i = pl.program_id(1)
    out_ref[0] = strip_ref[0, :, pl.ds(NJ - i, NJ), :]


def kernel(n, rel_bias_table):
    del n
    strips = pl.pallas_call(
        _build_body,
        grid=(H,),
        in_specs=[pl.BlockSpec(memory_space=pltpu.SMEM)],
        out_specs=pl.BlockSpec((1, BI, NBLK, 128), lambda h: (h, 0, 0, 0)),
        out_shape=jax.ShapeDtypeStruct((H, BI, NBLK, 128), jnp.float32),
        compiler_params=pltpu.CompilerParams(
            dimension_semantics=("parallel",),
        ),
    )(rel_bias_table)
    out = pl.pallas_call(
        _expand_body,
        grid=(H, N // BI),
        in_specs=[pl.BlockSpec((1, BI, NBLK, 128), lambda h, i: (h, 0, 0, 0))],
        out_specs=pl.BlockSpec((1, BI, NJ, 128), lambda h, i: (h, i, 0, 0)),
        out_shape=jax.ShapeDtypeStruct((H, N, NJ, 128), jnp.float32),
        compiler_params=pltpu.CompilerParams(
            dimension_semantics=("parallel", "arbitrary"),
        ),
    )(strips)
    return out.reshape(H, N, N)


# speedup vs baseline: 44.9745x; 1.3155x over previous
import math

import jax
import jax.numpy as jnp
from jax.experimental import pallas as pl
from jax.experimental.pallas import tpu as pltpu

N = 2048
H = 16
BI = 128
NJ = N // 128
NBLK = 2 * (NJ - 1) + 3
K = 8
STEPS = H * NJ


def _body(tbl_ref, out_ref, strip_ref, sems):
    h = pl.program_id(0)
    i = pl.program_id(1)
    s = h * NJ + i

    @pl.when(i == 0)
    def _build_master_strip():
        p = h % 2
        c_lo = tbl_ref[15, h]
        c_hi = tbl_ref[31, h]
        strip_ref[p, :, 0:NJ - 1, :] = jnp.full((BI, NJ - 1, 128), c_lo, jnp.float32)
        strip_ref[p, :, NJ + 2:NBLK, :] = jnp.full((BI, NJ - 1, 128), c_hi, jnp.float32)
        r = jax.lax.broadcasted_iota(jnp.int32, (BI, 3, 128), 0)
        t = jax.lax.broadcasted_iota(jnp.int32, (BI, 3, 128), 1)
        c = jax.lax.broadcasted_iota(jnp.int32, (BI, 3, 128), 2)
        rel = (c + (t - 1) * 128) - r
        nn = -rel
        ret = (nn < 0).astype(jnp.int32) * 16
        na = jnp.abs(nn)
        is_small = na < 8
        vl = 8 + (
            jnp.log(na.astype(jnp.float32) / 8.0) / math.log(16.0) * 8.0
        ).astype(jnp.int32)
        vl = jnp.minimum(vl, jnp.full_like(vl, 15))
        bucket = ret + jnp.where(is_small, na, vl)
        acc = jnp.zeros((BI, 3, 128), jnp.float32)
        for k in range(32):
            acc = acc + jnp.where(bucket == k, tbl_ref[k, h], 0.0)
        strip_ref[p, :, NJ - 1:NJ + 2, :] = acc

    def copy_for(step):
        hh = step // NJ
        ii = step % NJ
        return pltpu.make_async_copy(
            strip_ref.at[hh % 2, :, pl.ds(NJ - ii, NJ), :],
            out_ref.at[hh, pl.ds(ii * BI, BI), :, :],
            sems.at[step % K],
        )

    @pl.when(s >= K)
    def _wait_oldest():
        copy_for(s - K).wait()

    copy_for(s).start()

    @pl.when(s == STEPS - 1)
    def _drain():
        for d in range(K):
            copy_for(STEPS - K + d).wait()


def kernel(n, rel_bias_table):
    del n
    out = pl.pallas_call(
        _body,
        grid=(H, NJ),
        in_specs=[pl.BlockSpec(memory_space=pltpu.SMEM)],
        out_specs=pl.BlockSpec(memory_space=pl.ANY),
        out_shape=jax.ShapeDtypeStruct((H, N, NJ, 128), jnp.float32),
        scratch_shapes=[
            pltpu.VMEM((2, BI, NBLK, 128), jnp.float32),
            pltpu.SemaphoreType.DMA((K,)),
        ],
        compiler_params=pltpu.CompilerParams(
            dimension_semantics=("arbitrary", "arbitrary"),
        ),
    )(rel_bias_table)
    return out.reshape(H, N, N)
